# baseline (device time: 86336 ns/iter reference)
import jax
import jax.numpy as jnp
from jax import lax
from jax.experimental import pallas as pl
from jax.experimental.pallas import tpu as pltpu

N_DEV = 32
G = 4
NBLK = N_DEV // G


def kernel(x, w_mat, scale_x, scale_w):
    m_per, k = x.shape
    _, n = w_mat.shape
    n_per = n // N_DEV
    gcols = G * n_per

    def body(x_ref, w_ref, sx_ref, sw_ref, out_ref,
             xb_ref, wbuf_ref, comm_ref, copy_sems, send_sems, recv_sems):
        my = lax.axis_index("i")
        my_g = lax.div(my, G)
        scale = sx_ref[0] * sw_ref[0]
        xb_ref[:, :] = x_ref[:, :].astype(jnp.bfloat16)

        def wcopy(s):
            g = lax.rem(my_g + s, NBLK)
            return pltpu.make_async_copy(
                w_ref.at[:, pl.ds(g * gcols, gcols)],
                wbuf_ref.at[s % 2],
                copy_sems.at[s % 2],
            )

        wcopy(0).start()
        for s in range(NBLK):
            g = lax.rem(my_g + s, NBLK)
            if s + 1 < NBLK:
                wcopy(s + 1).start()
            wcopy(s).wait()
            blk = lax.dot_general(
                xb_ref[:, :], wbuf_ref[s % 2].astype(jnp.bfloat16),
                (((1,), (0,)), ((), ())),
                preferred_element_type=jnp.float32,
            ) * scale
            for j in range(G):
                t = g * G + j
                sub = blk[:, j * n_per:(j + 1) * n_per]
                slot = s * G + j
                comm_ref[slot, :, :] = sub

                def send_it():
                    rdma = pltpu.make_async_remote_copy(
                        src_ref=comm_ref.at[slot],
                        dst_ref=out_ref.at[pl.ds(my * m_per, m_per), :],
                        send_sem=send_sems.at[slot],
                        recv_sem=recv_sems.at[my],
                        device_id=(t,),
                        device_id_type=pl.DeviceIdType.MESH,
                    )
                    rdma.start()

                if s == 0:
                    @pl.when(t == my)
                    def _():
                        out_ref[pl.ds(my * m_per, m_per), :] = sub

                    @pl.when(t != my)
                    def _():
                        send_it()
                else:
                    send_it()

        for d in range(N_DEV):
            @pl.when(d != my)
            def _():
                recv = pltpu.make_async_remote_copy(
                    src_ref=comm_ref.at[0],
                    dst_ref=out_ref.at[pl.ds(d * m_per, m_per), :],
                    send_sem=send_sems.at[0],
                    recv_sem=recv_sems.at[d],
                    device_id=(d,),
                    device_id_type=pl.DeviceIdType.MESH,
                )
                recv.wait_recv()

        for slot in range(N_DEV):
            @pl.when(slot != lax.rem(my, G))
            def _():
                send = pltpu.make_async_remote_copy(
                    src_ref=comm_ref.at[slot],
                    dst_ref=out_ref.at[pl.ds(my * m_per, m_per), :],
                    send_sem=send_sems.at[slot],
                    recv_sem=recv_sems.at[my],
                    device_id=(0,),
                    device_id_type=pl.DeviceIdType.MESH,
                )
                send.wait_send()

    return pl.pallas_call(
        body,
        out_shape=jax.ShapeDtypeStruct((N_DEV * m_per, n_per), jnp.float32),
        in_specs=[
            pl.BlockSpec(memory_space=pltpu.VMEM),
            pl.BlockSpec(memory_space=pl.ANY),
            pl.BlockSpec(memory_space=pltpu.SMEM),
            pl.BlockSpec(memory_space=pltpu.SMEM),
        ],
        out_specs=pl.BlockSpec(memory_space=pltpu.VMEM),
        scratch_shapes=[
            pltpu.VMEM((m_per, k), jnp.bfloat16),
            pltpu.VMEM((2, k, gcols), jnp.float32),
            pltpu.VMEM((N_DEV, m_per, n_per), jnp.float32),
            pltpu.SemaphoreType.DMA((2,)),
            pltpu.SemaphoreType.DMA((N_DEV,)),
            pltpu.SemaphoreType.DMA((N_DEV,)),
        ],
        compiler_params=pltpu.CompilerParams(
            vmem_limit_bytes=56 * 1024 * 1024,
        ),
    )(x, w_mat, scale_x, scale_w)


# device time: 79420 ns/iter; 1.0871x vs baseline; 1.0871x over previous
import jax
import jax.numpy as jnp
from jax import lax
from jax.experimental import pallas as pl
from jax.experimental.pallas import tpu as pltpu

N_DEV = 32
DEPTH = 4


def kernel(x, w_mat, scale_x, scale_w):
    m_per, k = x.shape
    _, n = w_mat.shape
    n_per = n // N_DEV

    def body(x_ref, w_ref, sx_ref, sw_ref, out_ref,
             xb_ref, wbuf_ref, comm_ref, copy_sems, send_sems, recv_sems):
        my = lax.axis_index("i")
        scale = sx_ref[0] * sw_ref[0]
        xb_ref[:, :] = x_ref[:, :].astype(jnp.bfloat16)

        def wcopy(s):
            t = lax.rem(my + s, N_DEV)
            return pltpu.make_async_copy(
                w_ref.at[:, pl.ds(t * n_per, n_per)],
                wbuf_ref.at[s % DEPTH],
                copy_sems.at[s % DEPTH],
            )

        for s in range(DEPTH - 1):
            wcopy(s).start()
        for s in range(N_DEV):
            t = lax.rem(my + s, N_DEV)
            if s + DEPTH - 1 < N_DEV:
                wcopy(s + DEPTH - 1).start()
            wcopy(s).wait()
            blk = lax.dot_general(
                xb_ref[:, :], wbuf_ref[s % DEPTH].astype(jnp.bfloat16),
                (((1,), (0,)), ((), ())),
                preferred_element_type=jnp.float32,
            ) * scale
            if s == 0:
                out_ref[pl.ds(my * m_per, m_per), :] = blk
            else:
                comm_ref[s, :, :] = blk
                rdma = pltpu.make_async_remote_copy(
                    src_ref=comm_ref.at[s],
                    dst_ref=out_ref.at[pl.ds(my * m_per, m_per), :],
                    send_sem=send_sems.at[s],
                    recv_sem=recv_sems.at[s],
                    device_id=(t,),
                    device_id_type=pl.DeviceIdType.MESH,
                )
                rdma.start()

        for s in range(1, N_DEV):
            src = lax.rem(my - s + N_DEV, N_DEV)
            recv = pltpu.make_async_remote_copy(
                src_ref=comm_ref.at[s],
                dst_ref=out_ref.at[pl.ds(src * m_per, m_per), :],
                send_sem=send_sems.at[s],
                recv_sem=recv_sems.at[s],
                device_id=(src,),
                device_id_type=pl.DeviceIdType.MESH,
            )
            recv.wait_recv()

        for s in range(1, N_DEV):
            t = lax.rem(my + s, N_DEV)
            send = pltpu.make_async_remote_copy(
                src_ref=comm_ref.at[s],
                dst_ref=out_ref.at[pl.ds(my * m_per, m_per), :],
                send_sem=send_sems.at[s],
                recv_sem=recv_sems.at[s],
                device_id=(t,),
                device_id_type=pl.DeviceIdType.MESH,
            )
            send.wait_send()

    return pl.pallas_call(
        body,
        out_shape=jax.ShapeDtypeStruct((N_DEV * m_per, n_per), jnp.float32),
        in_specs=[
            pl.BlockSpec(memory_space=pltpu.VMEM),
            pl.BlockSpec(memory_space=pl.ANY),
            pl.BlockSpec(memory_space=pltpu.SMEM),
            pl.BlockSpec(memory_space=pltpu.SMEM),
        ],
        out_specs=pl.BlockSpec(memory_space=pltpu.VMEM),
        scratch_shapes=[
            pltpu.VMEM((m_per, k), jnp.bfloat16),
            pltpu.VMEM((DEPTH, k, n_per), jnp.float32),
            pltpu.VMEM((N_DEV, m_per, n_per), jnp.float32),
            pltpu.SemaphoreType.DMA((DEPTH,)),
            pltpu.SemaphoreType.DMA((N_DEV,)),
            pltpu.SemaphoreType.DMA((N_DEV,)),
        ],
        compiler_params=pltpu.CompilerParams(
            vmem_limit_bytes=56 * 1024 * 1024,
        ),
    )(x, w_mat, scale_x, scale_w)


# device time: 79163 ns/iter; 1.0906x vs baseline; 1.0032x over previous
import jax
import jax.numpy as jnp
from jax import lax
from jax.experimental import pallas as pl
from jax.experimental.pallas import tpu as pltpu

N_DEV = 32
DEPTH = 2


def kernel(x, w_mat, scale_x, scale_w):
    m_per, k = x.shape
    _, n = w_mat.shape
    n_per = n // N_DEV

    def body(x_ref, w_ref, sx_ref, sw_ref, out_ref,
             xb_ref, wbuf_ref, comm_ref, copy_sems, send_sems, recv_sems):
        my = lax.axis_index("i")
        scale = sx_ref[0] * sw_ref[0]
        xb_ref[:, :] = x_ref[:, :].astype(jnp.bfloat16)

        def wcopy(s):
            t = lax.rem(my + s, N_DEV)
            return pltpu.make_async_copy(
                w_ref.at[:, pl.ds(t * n_per, n_per)],
                wbuf_ref.at[s % DEPTH],
                copy_sems.at[s % DEPTH],
            )

        for s in range(DEPTH - 1):
            wcopy(s).start()
        for s in range(N_DEV):
            t = lax.rem(my + s, N_DEV)
            if s + DEPTH - 1 < N_DEV:
                wcopy(s + DEPTH - 1).start()
            wcopy(s).wait()
            blk = lax.dot_general(
                xb_ref[:, :], wbuf_ref[s % DEPTH].astype(jnp.bfloat16),
                (((1,), (0,)), ((), ())),
                preferred_element_type=jnp.float32,
            ) * scale
            if s == 0:
                out_ref[pl.ds(my * m_per, m_per), :] = blk
            else:
                comm_ref[s, :, :] = blk
                rdma = pltpu.make_async_remote_copy(
                    src_ref=comm_ref.at[s],
                    dst_ref=out_ref.at[pl.ds(my * m_per, m_per), :],
                    send_sem=send_sems.at[s],
                    recv_sem=recv_sems.at[s],
                    device_id=(t,),
                    device_id_type=pl.DeviceIdType.MESH,
                )
                rdma.start()

        for s in range(1, N_DEV):
            src = lax.rem(my - s + N_DEV, N_DEV)
            recv = pltpu.make_async_remote_copy(
                src_ref=comm_ref.at[s],
                dst_ref=out_ref.at[pl.ds(src * m_per, m_per), :],
                send_sem=send_sems.at[s],
                recv_sem=recv_sems.at[s],
                device_id=(src,),
                device_id_type=pl.DeviceIdType.MESH,
            )
            recv.wait_recv()

        for s in range(1, N_DEV):
            t = lax.rem(my + s, N_DEV)
            send = pltpu.make_async_remote_copy(
                src_ref=comm_ref.at[s],
                dst_ref=out_ref.at[pl.ds(my * m_per, m_per), :],
                send_sem=send_sems.at[s],
                recv_sem=recv_sems.at[s],
                device_id=(t,),
                device_id_type=pl.DeviceIdType.MESH,
            )
            send.wait_send()

    return pl.pallas_call(
        body,
        out_shape=jax.ShapeDtypeStruct((N_DEV * m_per, n_per), jnp.float32),
        in_specs=[
            pl.BlockSpec(memory_space=pltpu.VMEM),
            pl.BlockSpec(memory_space=pl.ANY),
            pl.BlockSpec(memory_space=pltpu.SMEM),
            pl.BlockSpec(memory_space=pltpu.SMEM),
        ],
        out_specs=pl.BlockSpec(memory_space=pltpu.VMEM),
        scratch_shapes=[
            pltpu.VMEM((m_per, k), jnp.bfloat16),
            pltpu.VMEM((DEPTH, k, n_per), jnp.float32),
            pltpu.VMEM((N_DEV, m_per, n_per), jnp.float32),
            pltpu.SemaphoreType.DMA((DEPTH,)),
            pltpu.SemaphoreType.DMA((N_DEV,)),
            pltpu.SemaphoreType.DMA((N_DEV,)),
        ],
        compiler_params=pltpu.CompilerParams(
            vmem_limit_bytes=56 * 1024 * 1024,
        ),
    )(x, w_mat, scale_x, scale_w)
